# baseline (speedup 1.0000x reference)
import functools

import jax
import jax.numpy as jnp
from jax import lax
from jax.experimental import pallas as pl
from jax.experimental.pallas import tpu as pltpu
from jax.experimental.pallas import tpu_sc as plsc

_L = 16
_K = 128
_NC = 2
_NS = 16
_NW = _NC * _NS


_NBUF = 4


def _deg_kernel(n, np_rows, ns0, ns1):
  mesh = plsc.VectorSubcoreMesh(core_axis_name="c", subcore_axis_name="s")
  nc0, nc1 = ns0 * _NBUF, ns1 * _NBUF
  nc_max = max(nc0, nc1)

  @functools.partial(
      pl.kernel, mesh=mesh,
      out_type=jax.ShapeDtypeStruct((_NC, n), jnp.float32),
      compiler_params=pltpu.CompilerParams(use_tc_tiling_on_sc=False),
      scratch_types=[
          pltpu.VMEM((nc_max, _K), jnp.int32),
          pltpu.VMEM((_K,), jnp.float32),
          pltpu.VMEM((np_rows,), jnp.float32),
          pltpu.VMEM_SHARED((np_rows,), jnp.float32),
          pltpu.SemaphoreType.DMA,
      ] + [pltpu.SemaphoreType.DMA] * _NBUF)
  def degk(dst_hbm, out_hbm, idx_d, ones_v, zbuf, acc, sem_i, *sem_sc):
    c = lax.axis_index("c")
    s = lax.axis_index("s")

    @pl.when(c == 0)
    def _stage0():
      pltpu.async_copy(dst_hbm.at[pl.ds(s * nc0, nc0), :],
                       idx_d.at[pl.ds(0, nc0), :], sem_i)

    @pl.when(c == 1)
    def _stage1():
      pltpu.async_copy(dst_hbm.at[pl.ds(_NS * nc0 + s * nc1, nc1), :],
                       idx_d.at[pl.ds(0, nc1), :], sem_i)

    for j in range(_K // _L):
      ones_v[pl.ds(j * _L, _L)] = jnp.ones((_L,), jnp.float32)

    @pl.when(s == 0)
    def _zero():
      def zi(i, carry):
        zbuf[pl.ds(i * _L, _L)] = jnp.zeros((_L,), jnp.float32)
        return carry
      lax.fori_loop(0, np_rows // _L, zi, 0)
      pltpu.sync_copy(zbuf, acc)

    @pl.when(c == 0)
    def _wait0():
      pltpu.make_async_copy(dst_hbm.at[pl.ds(s * nc0, nc0), :],
                            idx_d.at[pl.ds(0, nc0), :], sem_i).wait()

    @pl.when(c == 1)
    def _wait1():
      pltpu.make_async_copy(dst_hbm.at[pl.ds(0, nc1), :],
                            idx_d.at[pl.ds(0, nc1), :], sem_i).wait()
    plsc.subcore_barrier()

    n_super = jnp.where(c == 0, ns0, ns1)

    def body(gs, carry):
      for b in range(_NBUF):
        g = gs * _NBUF + b

        @pl.when(gs > 0)
        def _drain():
          pltpu.make_async_copy(ones_v, acc.at[idx_d.at[g]],
                                sem_sc[b]).wait()
        pltpu.async_copy(ones_v, acc.at[idx_d.at[g]], sem_sc[b], add=True)
      return carry
    lax.fori_loop(0, n_super, body, 0)
    for b in range(_NBUF):
      g = (n_super - 1) * _NBUF + b
      pltpu.make_async_copy(ones_v, acc.at[idx_d.at[g]], sem_sc[b]).wait()

    plsc.subcore_barrier()

    @pl.when(s == 0)
    def _out():
      pltpu.sync_copy(acc.at[pl.ds(0, n)], out_hbm.at[c])

  return degk


def _msg_kernel(n, np_rows, d, sup0, sup1):
  mesh = plsc.VectorSubcoreMesh(core_axis_name="c", subcore_axis_name="s")
  zrows = np_rows // _NS
  orows = n // _NS
  zspan = _NBUF * _K

  @functools.partial(
      pl.kernel, mesh=mesh,
      out_type=jax.ShapeDtypeStruct((_NC, n, d), jnp.float32),
      compiler_params=pltpu.CompilerParams(use_tc_tiling_on_sc=False),
      scratch_types=[
          pltpu.VMEM((2, _NBUF, _K), jnp.int32),
          pltpu.VMEM((2, _NBUF, _K), jnp.int32),
          pltpu.VMEM((2, _NBUF, _K, d), jnp.float32),
          pltpu.VMEM_SHARED((np_rows, d), jnp.float32),
          pltpu.SMEM((1,), jnp.int32),
          pltpu.SemaphoreType.DMA,
          pltpu.SemaphoreType.DMA,
      ] + [pltpu.SemaphoreType.DMA] * (4 * _NBUF))
  def msgk(y_hbm, src_hbm, dst_hbm, out_hbm, idx_s, idx_d, rows_v, acc, cnt,
           *sems):
    sem_i = sems[:2]
    sem_g = (sems[2:2 + _NBUF], sems[2 + _NBUF:2 + 2 * _NBUF])
    sem_sc = (sems[2 + 2 * _NBUF:2 + 3 * _NBUF], sems[2 + 3 * _NBUF:])
    c = lax.axis_index("c")
    s = lax.axis_index("s")
    n_sup = jnp.where(c == 0, sup0, sup1)
    shard0 = jnp.where(c == 0, 0, sup0)

    @pl.when(s == 0)
    def _cinit():
      cnt[0] = 0

    with jax.named_scope("zfill"):
      def zi(i, carry):
        for j in range(d // _L):
          rows_v[0, i // _K, i % _K, pl.ds(j * _L, _L)] = (
              jnp.zeros((_L,), jnp.float32))
        return carry
      lax.fori_loop(0, min(zspan, zrows), zi, 0)
    with jax.named_scope("zdma"):
      done = 0
      while done < zrows:
        step = min(_K, zrows - done)
        pltpu.sync_copy(
            rows_v.at[0, done // _K % _NBUF, pl.ds(0, step), :],
            acc.at[pl.ds(s * zrows + done, step), :])
        done += step
    plsc.subcore_barrier()

    def fire_idx(p, sup):
      chunk0 = (shard0 + sup) * _NBUF
      pltpu.async_copy(src_hbm.at[pl.ds(chunk0, _NBUF), :], idx_s.at[p],
                       sem_i[p])
      pltpu.async_copy(dst_hbm.at[pl.ds(chunk0, _NBUF), :], idx_d.at[p],
                       sem_i[p])

    def wait_idx(p):
      pltpu.make_async_copy(src_hbm.at[pl.ds(0, _NBUF), :], idx_s.at[p],
                            sem_i[p]).wait()
      pltpu.make_async_copy(dst_hbm.at[pl.ds(0, _NBUF), :], idx_d.at[p],
                            sem_i[p]).wait()

    with jax.named_scope("edges"):
      participate = jnp.logical_or(c == 0, s < 14)
      s_init = []
      for p in range(2):
        sp = lax.cond(
            participate,
            lambda: plsc.fetch_and_add(cnt.at[0], 1, subcore_id=0),
            lambda: n_sup)

        @pl.when(sp < n_sup)
        def _prime(p=p, sp=sp):
          fire_idx(p, sp)
        s_init.append(sp)

      def body(_, carry):
        new = []
        for p in range(2):
          sp = carry[p]
          active = sp < n_sup

          @pl.when(active)
          def _visit(p=p):
            wait_idx(p)
            for b in range(_NBUF):
              pltpu.async_copy(y_hbm.at[idx_s.at[p, b]], rows_v.at[p, b],
                               sem_g[p][b])
            for b in range(_NBUF):
              pltpu.make_async_copy(y_hbm.at[idx_s.at[p, b]],
                                    rows_v.at[p, b], sem_g[p][b]).wait()
              pltpu.async_copy(rows_v.at[p, b], acc.at[idx_d.at[p, b]],
                               sem_sc[p][b], add=True)
            for b in range(_NBUF):
              pltpu.make_async_copy(rows_v.at[p, b], acc.at[idx_d.at[p, b]],
                                    sem_sc[p][b]).wait()
          nxt = lax.cond(
              active,
              lambda: plsc.fetch_and_add(cnt.at[0], 1, subcore_id=0),
              lambda sp=sp: sp)

          @pl.when(active & (nxt < n_sup))
          def _prefetch(p=p, nxt=nxt):
            fire_idx(p, nxt)
          new.append(nxt)
        return tuple(new)
      lax.fori_loop(0, n_sup, body, tuple(s_init))

    with jax.named_scope("endbar"):
      plsc.subcore_barrier()
    with jax.named_scope("copyout"):
      pltpu.sync_copy(acc.at[pl.ds(s * orows, orows), :],
                      out_hbm.at[c, pl.ds(s * orows, orows), :])

  return msgk


def _dis(deg_parts, n):
  def body(p_ref, dis_ref):
    dis_ref[...] = lax.rsqrt(1.0 + p_ref[0:1, :] + p_ref[1:2, :])
  return pl.pallas_call(
      body, out_shape=jax.ShapeDtypeStruct((1, n), jnp.float32))(deg_parts)


def _scale_matmul(x, w, dis_col, bn):
  n, k = x.shape
  d = w.shape[1]

  def body(x_ref, w_ref, dis_ref, y_ref):
    y_ref[...] = dis_ref[...] * jnp.dot(
        x_ref[...], w_ref[...], preferred_element_type=jnp.float32)

  return pl.pallas_call(
      body,
      grid=(n // bn,),
      in_specs=[
          pl.BlockSpec((bn, k), lambda i: (i, 0)),
          pl.BlockSpec((k, d), lambda i: (0, 0)),
          pl.BlockSpec((bn, 1), lambda i: (i, 0)),
      ],
      out_specs=pl.BlockSpec((bn, d), lambda i: (i, 0)),
      out_shape=jax.ShapeDtypeStruct((n, d), jnp.float32),
  )(x, w, dis_col)


def _mid_layer(s_parts, y1, dis_col, b1, w2, bn):
  n, d1 = y1.shape
  d2 = w2.shape[1]

  def body(s_ref, y1_ref, dis_ref, b1_ref, w2_ref, y2_ref):
    t = s_ref[0] + s_ref[1] + y1_ref[...]
    h = jnp.maximum(dis_ref[...] * t + b1_ref[...], 0.0)
    y2_ref[...] = dis_ref[...] * jnp.dot(
        h, w2_ref[...], preferred_element_type=jnp.float32)

  return pl.pallas_call(
      body,
      grid=(n // bn,),
      in_specs=[
          pl.BlockSpec((_NC, bn, d1), lambda i: (0, i, 0)),
          pl.BlockSpec((bn, d1), lambda i: (i, 0)),
          pl.BlockSpec((bn, 1), lambda i: (i, 0)),
          pl.BlockSpec((1, d1), lambda i: (0, 0)),
          pl.BlockSpec((d1, d2), lambda i: (0, 0)),
      ],
      out_specs=pl.BlockSpec((bn, d2), lambda i: (i, 0)),
      out_shape=jax.ShapeDtypeStruct((n, d2), jnp.float32),
  )(s_parts, y1, dis_col, b1, w2)


def _final_layer(s_parts, y2, dis_col, b2, bn):
  n, d2 = y2.shape

  def body(s_ref, y2_ref, dis_ref, b2_ref, z_ref):
    z_ref[...] = dis_ref[...] * (s_ref[0] + s_ref[1] + y2_ref[...]) \
        + b2_ref[...]

  return pl.pallas_call(
      body,
      grid=(n // bn,),
      in_specs=[
          pl.BlockSpec((_NC, bn, d2), lambda i: (0, i, 0)),
          pl.BlockSpec((bn, d2), lambda i: (i, 0)),
          pl.BlockSpec((bn, 1), lambda i: (i, 0)),
          pl.BlockSpec((1, d2), lambda i: (0, 0)),
      ],
      out_specs=pl.BlockSpec((bn, d2), lambda i: (i, 0)),
      out_shape=jax.ShapeDtypeStruct((n, d2), jnp.float32),
  )(s_parts, y2, dis_col, b2)


@jax.jit
def kernel(x, edge_index, W1, b1, W2, b2):
  n = x.shape[0]
  e = edge_index.shape[1]
  src = edge_index[0].astype(jnp.int32)
  dst = edge_index[1].astype(jnp.int32)

  align = _NS * _K * _NBUF
  n_super_tot = -(-e // align)
  e_pad = n_super_tot * align
  np_rows = n + _L
  if e_pad > e:
    src = jnp.concatenate([src, jnp.zeros((e_pad - e,), jnp.int32)])
    dst = jnp.concatenate([dst, jnp.full((e_pad - e,), n, jnp.int32)])
  src = src.reshape(e_pad // _K, _K)
  dst = dst.reshape(e_pad // _K, _K)

  sp_deg = (-(-n_super_tot * 27) // 40, None)
  sp_deg = (sp_deg[0], n_super_tot - sp_deg[0])
  sup_tot = n_super_tot * _NS
  sup0 = (sup_tot * 53 + 50) // 100
  sp_msg = (sup0, sup_tot - sup0)

  bn = 2000
  deg_parts = _deg_kernel(n, np_rows, *sp_deg)(dst)
  dis_col = _dis(deg_parts, n).reshape(n, 1)

  y1 = _scale_matmul(x, W1, dis_col, bn)
  s1 = _msg_kernel(n, np_rows, y1.shape[1], *sp_msg)(y1, src, dst)
  y2 = _mid_layer(s1, y1, dis_col, b1.reshape(1, -1), W2, bn)
  s2 = _msg_kernel(n, np_rows, y2.shape[1], *sp_msg)(y2, src, dst)
  return _final_layer(s2, y2, dis_col, b2.reshape(1, -1), bn)


# speedup vs baseline: 1.0677x; 1.0677x over previous
import functools

import jax
import jax.numpy as jnp
from jax import lax
from jax.experimental import pallas as pl
from jax.experimental.pallas import tpu as pltpu
from jax.experimental.pallas import tpu_sc as plsc

_L = 16
_K = 128
_NC = 2
_NS = 16
_NW = _NC * _NS


_NBUF = 4


def _deg_kernel(n, np_rows, ns0, ns1):
  mesh = plsc.VectorSubcoreMesh(core_axis_name="c", subcore_axis_name="s")
  nc0, nc1 = ns0 * _NBUF, ns1 * _NBUF
  nc_max = max(nc0, nc1)

  @functools.partial(
      pl.kernel, mesh=mesh,
      out_type=jax.ShapeDtypeStruct((_NC, n), jnp.float32),
      compiler_params=pltpu.CompilerParams(use_tc_tiling_on_sc=False),
      scratch_types=[
          pltpu.VMEM((nc_max, _K), jnp.int32),
          pltpu.VMEM((_K,), jnp.float32),
          pltpu.VMEM((np_rows,), jnp.float32),
          pltpu.VMEM_SHARED((np_rows,), jnp.float32),
          pltpu.SemaphoreType.DMA,
      ] + [pltpu.SemaphoreType.DMA] * _NBUF)
  def degk(dst_hbm, out_hbm, idx_d, ones_v, zbuf, acc, sem_i, *sem_sc):
    c = lax.axis_index("c")
    s = lax.axis_index("s")

    @pl.when(c == 0)
    def _stage0():
      pltpu.async_copy(dst_hbm.at[pl.ds(s * nc0, nc0), :],
                       idx_d.at[pl.ds(0, nc0), :], sem_i)

    @pl.when(c == 1)
    def _stage1():
      pltpu.async_copy(dst_hbm.at[pl.ds(_NS * nc0 + s * nc1, nc1), :],
                       idx_d.at[pl.ds(0, nc1), :], sem_i)

    for j in range(_K // _L):
      ones_v[pl.ds(j * _L, _L)] = jnp.ones((_L,), jnp.float32)

    @pl.when(s == 0)
    def _zero():
      def zi(i, carry):
        zbuf[pl.ds(i * _L, _L)] = jnp.zeros((_L,), jnp.float32)
        return carry
      lax.fori_loop(0, np_rows // _L, zi, 0)
      pltpu.sync_copy(zbuf, acc)

    @pl.when(c == 0)
    def _wait0():
      pltpu.make_async_copy(dst_hbm.at[pl.ds(s * nc0, nc0), :],
                            idx_d.at[pl.ds(0, nc0), :], sem_i).wait()

    @pl.when(c == 1)
    def _wait1():
      pltpu.make_async_copy(dst_hbm.at[pl.ds(0, nc1), :],
                            idx_d.at[pl.ds(0, nc1), :], sem_i).wait()
    plsc.subcore_barrier()

    n_super = jnp.where(c == 0, ns0, ns1)

    def body(gs, carry):
      for b in range(_NBUF):
        g = gs * _NBUF + b

        @pl.when(gs > 0)
        def _drain():
          pltpu.make_async_copy(ones_v, acc.at[idx_d.at[g]],
                                sem_sc[b]).wait()
        pltpu.async_copy(ones_v, acc.at[idx_d.at[g]], sem_sc[b], add=True)
      return carry
    lax.fori_loop(0, n_super, body, 0)
    for b in range(_NBUF):
      g = (n_super - 1) * _NBUF + b
      pltpu.make_async_copy(ones_v, acc.at[idx_d.at[g]], sem_sc[b]).wait()

    plsc.subcore_barrier()

    @pl.when(s == 0)
    def _out():
      pltpu.sync_copy(acc.at[pl.ds(0, n)], out_hbm.at[c])

  return degk


def _msg_kernel(n, np_rows, d, sup0, sup1):
  mesh = plsc.VectorSubcoreMesh(core_axis_name="c", subcore_axis_name="s")
  zrows = np_rows // _NS
  orows = n // _NS
  zspan = _NBUF * _K

  @functools.partial(
      pl.kernel, mesh=mesh,
      out_type=jax.ShapeDtypeStruct((_NC, n, d), jnp.float32),
      compiler_params=pltpu.CompilerParams(use_tc_tiling_on_sc=False),
      scratch_types=[
          pltpu.VMEM((2, _NBUF, _K), jnp.int32),
          pltpu.VMEM((2, _NBUF, _K), jnp.int32),
          pltpu.VMEM((2, _NBUF, _K, d), jnp.float32),
          pltpu.VMEM_SHARED((np_rows, d), jnp.float32),
          pltpu.SMEM((1,), jnp.int32),
          pltpu.SemaphoreType.DMA,
          pltpu.SemaphoreType.DMA,
      ] + [pltpu.SemaphoreType.DMA] * (4 * _NBUF))
  def msgk(y_hbm, src_hbm, dst_hbm, out_hbm, idx_s, idx_d, rows_v, acc, cnt,
           *sems):
    sem_i = sems[:2]
    sem_g = (sems[2:2 + _NBUF], sems[2 + _NBUF:2 + 2 * _NBUF])
    sem_sc = (sems[2 + 2 * _NBUF:2 + 3 * _NBUF], sems[2 + 3 * _NBUF:])
    c = lax.axis_index("c")
    s = lax.axis_index("s")
    n_sup = jnp.where(c == 0, sup0, sup1)
    shard0 = jnp.where(c == 0, 0, sup0)

    @pl.when(s == 0)
    def _cinit():
      cnt[0] = 0

    with jax.named_scope("zfill"):
      def zi(i, carry):
        for j in range(d // _L):
          rows_v[0, i // _K, i % _K, pl.ds(j * _L, _L)] = (
              jnp.zeros((_L,), jnp.float32))
        return carry
      lax.fori_loop(0, min(zspan, zrows), zi, 0)
    with jax.named_scope("zdma"):
      done = 0
      while done < zrows:
        step = min(_K, zrows - done)
        pltpu.sync_copy(
            rows_v.at[0, done // _K % _NBUF, pl.ds(0, step), :],
            acc.at[pl.ds(s * zrows + done, step), :])
        done += step
    plsc.subcore_barrier()

    def fire_idx(p, sup):
      chunk0 = (shard0 + sup) * _NBUF
      pltpu.async_copy(src_hbm.at[pl.ds(chunk0, _NBUF), :], idx_s.at[p],
                       sem_i[p])
      pltpu.async_copy(dst_hbm.at[pl.ds(chunk0, _NBUF), :], idx_d.at[p],
                       sem_i[p])

    def wait_idx(p):
      pltpu.make_async_copy(src_hbm.at[pl.ds(0, _NBUF), :], idx_s.at[p],
                            sem_i[p]).wait()
      pltpu.make_async_copy(dst_hbm.at[pl.ds(0, _NBUF), :], idx_d.at[p],
                            sem_i[p]).wait()

    with jax.named_scope("edges"):
      s_init = []
      for p in range(2):
        sp = plsc.fetch_and_add(cnt.at[0], 1, subcore_id=0)

        @pl.when(sp < n_sup)
        def _prime(p=p, sp=sp):
          fire_idx(p, sp)
        s_init.append(sp)

      def body(_, carry):
        new = []
        for p in range(2):
          sp = carry[p]
          active = sp < n_sup

          @pl.when(active)
          def _visit(p=p):
            wait_idx(p)
            for b in range(_NBUF):
              pltpu.async_copy(y_hbm.at[idx_s.at[p, b]], rows_v.at[p, b],
                               sem_g[p][b])
            for b in range(_NBUF):
              pltpu.make_async_copy(y_hbm.at[idx_s.at[p, b]],
                                    rows_v.at[p, b], sem_g[p][b]).wait()
              pltpu.async_copy(rows_v.at[p, b], acc.at[idx_d.at[p, b]],
                               sem_sc[p][b], add=True)
            for b in range(_NBUF):
              pltpu.make_async_copy(rows_v.at[p, b], acc.at[idx_d.at[p, b]],
                                    sem_sc[p][b]).wait()
          nxt = lax.cond(
              active,
              lambda: plsc.fetch_and_add(cnt.at[0], 1, subcore_id=0),
              lambda sp=sp: sp)

          @pl.when(active & (nxt < n_sup))
          def _prefetch(p=p, nxt=nxt):
            fire_idx(p, nxt)
          new.append(nxt)
        return tuple(new)
      lax.fori_loop(0, n_sup // 2 + 2, body, tuple(s_init))

    with jax.named_scope("endbar"):
      plsc.subcore_barrier()
    with jax.named_scope("copyout"):
      pltpu.sync_copy(acc.at[pl.ds(s * orows, orows), :],
                      out_hbm.at[c, pl.ds(s * orows, orows), :])

  return msgk


def _dis(deg_parts, n):
  def body(p_ref, dis_ref):
    dis_ref[...] = lax.rsqrt(1.0 + p_ref[0:1, :] + p_ref[1:2, :])
  return pl.pallas_call(
      body, out_shape=jax.ShapeDtypeStruct((1, n), jnp.float32))(deg_parts)


def _scale_matmul(x, w, dis_col, bn):
  n, k = x.shape
  d = w.shape[1]

  def body(x_ref, w_ref, dis_ref, y_ref):
    y_ref[...] = dis_ref[...] * jnp.dot(
        x_ref[...], w_ref[...], preferred_element_type=jnp.float32)

  return pl.pallas_call(
      body,
      grid=(n // bn,),
      in_specs=[
          pl.BlockSpec((bn, k), lambda i: (i, 0)),
          pl.BlockSpec((k, d), lambda i: (0, 0)),
          pl.BlockSpec((bn, 1), lambda i: (i, 0)),
      ],
      out_specs=pl.BlockSpec((bn, d), lambda i: (i, 0)),
      out_shape=jax.ShapeDtypeStruct((n, d), jnp.float32),
  )(x, w, dis_col)


def _mid_layer(s_parts, y1, dis_col, b1, w2, bn):
  n, d1 = y1.shape
  d2 = w2.shape[1]

  def body(s_ref, y1_ref, dis_ref, b1_ref, w2_ref, y2_ref):
    t = s_ref[0] + s_ref[1] + y1_ref[...]
    h = jnp.maximum(dis_ref[...] * t + b1_ref[...], 0.0)
    y2_ref[...] = dis_ref[...] * jnp.dot(
        h, w2_ref[...], preferred_element_type=jnp.float32)

  return pl.pallas_call(
      body,
      grid=(n // bn,),
      in_specs=[
          pl.BlockSpec((_NC, bn, d1), lambda i: (0, i, 0)),
          pl.BlockSpec((bn, d1), lambda i: (i, 0)),
          pl.BlockSpec((bn, 1), lambda i: (i, 0)),
          pl.BlockSpec((1, d1), lambda i: (0, 0)),
          pl.BlockSpec((d1, d2), lambda i: (0, 0)),
      ],
      out_specs=pl.BlockSpec((bn, d2), lambda i: (i, 0)),
      out_shape=jax.ShapeDtypeStruct((n, d2), jnp.float32),
  )(s_parts, y1, dis_col, b1, w2)


def _final_layer(s_parts, y2, dis_col, b2, bn):
  n, d2 = y2.shape

  def body(s_ref, y2_ref, dis_ref, b2_ref, z_ref):
    z_ref[...] = dis_ref[...] * (s_ref[0] + s_ref[1] + y2_ref[...]) \
        + b2_ref[...]

  return pl.pallas_call(
      body,
      grid=(n // bn,),
      in_specs=[
          pl.BlockSpec((_NC, bn, d2), lambda i: (0, i, 0)),
          pl.BlockSpec((bn, d2), lambda i: (i, 0)),
          pl.BlockSpec((bn, 1), lambda i: (i, 0)),
          pl.BlockSpec((1, d2), lambda i: (0, 0)),
      ],
      out_specs=pl.BlockSpec((bn, d2), lambda i: (i, 0)),
      out_shape=jax.ShapeDtypeStruct((n, d2), jnp.float32),
  )(s_parts, y2, dis_col, b2)


@jax.jit
def kernel(x, edge_index, W1, b1, W2, b2):
  n = x.shape[0]
  e = edge_index.shape[1]
  src = edge_index[0].astype(jnp.int32)
  dst = edge_index[1].astype(jnp.int32)

  align = _NS * _K * _NBUF
  n_super_tot = -(-e // align)
  e_pad = n_super_tot * align
  np_rows = n + _L
  if e_pad > e:
    src = jnp.concatenate([src, jnp.zeros((e_pad - e,), jnp.int32)])
    dst = jnp.concatenate([dst, jnp.full((e_pad - e,), n, jnp.int32)])
  src = src.reshape(e_pad // _K, _K)
  dst = dst.reshape(e_pad // _K, _K)

  sp_deg = (-(-n_super_tot * 31) // 40, None)
  sp_deg = (sp_deg[0], n_super_tot - sp_deg[0])
  sup_tot = n_super_tot * _NS
  sup0_d1 = (sup_tot * 72 + 50) // 100
  sup0_d2 = (sup_tot * 69 + 50) // 100
  sp_d1 = (sup0_d1, sup_tot - sup0_d1)
  sp_d2 = (sup0_d2, sup_tot - sup0_d2)

  bn = 2000
  deg_parts = _deg_kernel(n, np_rows, *sp_deg)(dst)
  dis_col = _dis(deg_parts, n).reshape(n, 1)

  y1 = _scale_matmul(x, W1, dis_col, bn)
  s1 = _msg_kernel(n, np_rows, y1.shape[1], *sp_d1)(y1, src, dst)
  y2 = _mid_layer(s1, y1, dis_col, b1.reshape(1, -1), W2, bn)
  s2 = _msg_kernel(n, np_rows, y2.shape[1], *sp_d2)(y2, src, dst)
  return _final_layer(s2, y2, dis_col, b2.reshape(1, -1), bn)


# speedup vs baseline: 1.0685x; 1.0007x over previous
import functools

import jax
import jax.numpy as jnp
from jax import lax
from jax.experimental import pallas as pl
from jax.experimental.pallas import tpu as pltpu
from jax.experimental.pallas import tpu_sc as plsc

_L = 16
_K = 128
_NC = 2
_NS = 16
_NW = _NC * _NS


_NBUF = 4


def _deg_kernel(n, np_rows, ns0, ns1):
  mesh = plsc.VectorSubcoreMesh(core_axis_name="c", subcore_axis_name="s")
  nc0, nc1 = ns0 * _NBUF, ns1 * _NBUF
  nc_max = max(nc0, nc1)

  @functools.partial(
      pl.kernel, mesh=mesh,
      out_type=jax.ShapeDtypeStruct((_NC, n), jnp.float32),
      compiler_params=pltpu.CompilerParams(use_tc_tiling_on_sc=False),
      scratch_types=[
          pltpu.VMEM((nc_max, _K), jnp.int32),
          pltpu.VMEM((_K,), jnp.float32),
          pltpu.VMEM((np_rows,), jnp.float32),
          pltpu.VMEM_SHARED((np_rows,), jnp.float32),
          pltpu.SemaphoreType.DMA,
      ] + [pltpu.SemaphoreType.DMA] * _NBUF)
  def degk(dst_hbm, out_hbm, idx_d, ones_v, zbuf, acc, sem_i, *sem_sc):
    c = lax.axis_index("c")
    s = lax.axis_index("s")

    @pl.when(c == 0)
    def _stage0():
      pltpu.async_copy(dst_hbm.at[pl.ds(s * nc0, nc0), :],
                       idx_d.at[pl.ds(0, nc0), :], sem_i)

    @pl.when(c == 1)
    def _stage1():
      pltpu.async_copy(dst_hbm.at[pl.ds(_NS * nc0 + s * nc1, nc1), :],
                       idx_d.at[pl.ds(0, nc1), :], sem_i)

    for j in range(_K // _L):
      ones_v[pl.ds(j * _L, _L)] = jnp.ones((_L,), jnp.float32)

    @pl.when(s == 0)
    def _zero():
      def zi(i, carry):
        zbuf[pl.ds(i * _L, _L)] = jnp.zeros((_L,), jnp.float32)
        return carry
      lax.fori_loop(0, np_rows // _L, zi, 0)
      pltpu.sync_copy(zbuf, acc)

    @pl.when(c == 0)
    def _wait0():
      pltpu.make_async_copy(dst_hbm.at[pl.ds(s * nc0, nc0), :],
                            idx_d.at[pl.ds(0, nc0), :], sem_i).wait()

    @pl.when(c == 1)
    def _wait1():
      pltpu.make_async_copy(dst_hbm.at[pl.ds(0, nc1), :],
                            idx_d.at[pl.ds(0, nc1), :], sem_i).wait()
    plsc.subcore_barrier()

    n_super = jnp.where(c == 0, ns0, ns1)

    def body(gs, carry):
      for b in range(_NBUF):
        g = gs * _NBUF + b

        @pl.when(gs > 0)
        def _drain():
          pltpu.make_async_copy(ones_v, acc.at[idx_d.at[g]],
                                sem_sc[b]).wait()
        pltpu.async_copy(ones_v, acc.at[idx_d.at[g]], sem_sc[b], add=True)
      return carry
    lax.fori_loop(0, n_super, body, 0)
    for b in range(_NBUF):
      g = (n_super - 1) * _NBUF + b
      pltpu.make_async_copy(ones_v, acc.at[idx_d.at[g]], sem_sc[b]).wait()

    plsc.subcore_barrier()

    @pl.when(s == 0)
    def _out():
      pltpu.sync_copy(acc.at[pl.ds(0, n)], out_hbm.at[c])

  return degk


def _msg_kernel(n, np_rows, d, sup0, sup1):
  mesh = plsc.VectorSubcoreMesh(core_axis_name="c", subcore_axis_name="s")
  zrows = np_rows // _NS
  orows = n // _NS
  zspan = _NBUF * _K

  @functools.partial(
      pl.kernel, mesh=mesh,
      out_type=jax.ShapeDtypeStruct((_NC, n, d), jnp.float32),
      compiler_params=pltpu.CompilerParams(use_tc_tiling_on_sc=False),
      scratch_types=[
          pltpu.VMEM((2, _NBUF, _K), jnp.int32),
          pltpu.VMEM((2, _NBUF, _K), jnp.int32),
          pltpu.VMEM((2, _NBUF, _K, d), jnp.float32),
          pltpu.VMEM_SHARED((np_rows, d), jnp.float32),
          pltpu.SMEM((1,), jnp.int32),
          pltpu.SemaphoreType.DMA,
          pltpu.SemaphoreType.DMA,
      ] + [pltpu.SemaphoreType.DMA] * (4 * _NBUF))
  def msgk(y_hbm, src_hbm, dst_hbm, out_hbm, idx_s, idx_d, rows_v, acc, cnt,
           *sems):
    sem_i = sems[:2]
    sem_g = (sems[2:2 + _NBUF], sems[2 + _NBUF:2 + 2 * _NBUF])
    sem_sc = (sems[2 + 2 * _NBUF:2 + 3 * _NBUF], sems[2 + 3 * _NBUF:])
    c = lax.axis_index("c")
    s = lax.axis_index("s")
    n_sup = jnp.where(c == 0, sup0, sup1)
    shard0 = jnp.where(c == 0, 0, sup0)

    @pl.when(s == 0)
    def _cinit():
      cnt[0] = 0

    with jax.named_scope("zfill"):
      def zi(i, carry):
        for j in range(d // _L):
          rows_v[0, i // _K, i % _K, pl.ds(j * _L, _L)] = (
              jnp.zeros((_L,), jnp.float32))
        return carry
      lax.fori_loop(0, min(zspan, zrows), zi, 0)
    with jax.named_scope("zdma"):
      done = 0
      while done < zrows:
        step = min(_K, zrows - done)
        pltpu.sync_copy(
            rows_v.at[0, done // _K % _NBUF, pl.ds(0, step), :],
            acc.at[pl.ds(s * zrows + done, step), :])
        done += step
    plsc.subcore_barrier()

    def fire_idx(p, sup):
      chunk0 = (shard0 + sup) * _NBUF
      pltpu.async_copy(src_hbm.at[pl.ds(chunk0, _NBUF), :], idx_s.at[p],
                       sem_i[p])
      pltpu.async_copy(dst_hbm.at[pl.ds(chunk0, _NBUF), :], idx_d.at[p],
                       sem_i[p])

    def wait_idx(p):
      pltpu.make_async_copy(src_hbm.at[pl.ds(0, _NBUF), :], idx_s.at[p],
                            sem_i[p]).wait()
      pltpu.make_async_copy(dst_hbm.at[pl.ds(0, _NBUF), :], idx_d.at[p],
                            sem_i[p]).wait()

    with jax.named_scope("edges"):
      s_init = []
      for p in range(2):
        sp = plsc.fetch_and_add(cnt.at[0], 1, subcore_id=0)

        @pl.when(sp < n_sup)
        def _prime(p=p, sp=sp):
          fire_idx(p, sp)
        s_init.append(sp)

      def body(_, carry):
        new = []
        for p in range(2):
          sp = carry[p]
          active = sp < n_sup

          @pl.when(active)
          def _visit(p=p):
            with jax.named_scope("vw"):
              wait_idx(p)
              for b in range(_NBUF):
                pltpu.async_copy(y_hbm.at[idx_s.at[p, b]], rows_v.at[p, b],
                                 sem_g[p][b])
            with jax.named_scope("vg"):
              for b in range(_NBUF):
                pltpu.make_async_copy(y_hbm.at[idx_s.at[p, b]],
                                      rows_v.at[p, b], sem_g[p][b]).wait()
                pltpu.async_copy(rows_v.at[p, b], acc.at[idx_d.at[p, b]],
                                 sem_sc[p][b], add=True)
            with jax.named_scope("vd"):
              for b in range(_NBUF):
                pltpu.make_async_copy(rows_v.at[p, b],
                                      acc.at[idx_d.at[p, b]],
                                      sem_sc[p][b]).wait()
          nxt = lax.cond(
              active,
              lambda: plsc.fetch_and_add(cnt.at[0], 1, subcore_id=0),
              lambda sp=sp: sp)

          @pl.when(active & (nxt < n_sup))
          def _prefetch(p=p, nxt=nxt):
            fire_idx(p, nxt)
          new.append(nxt)
        return tuple(new)
      lax.fori_loop(0, n_sup // 2 + 2, body, tuple(s_init))

    with jax.named_scope("endbar"):
      plsc.subcore_barrier()
    with jax.named_scope("copyout"):
      pltpu.sync_copy(acc.at[pl.ds(s * orows, orows), :],
                      out_hbm.at[c, pl.ds(s * orows, orows), :])

  return msgk


def _dis(deg_parts, n):
  def body(p_ref, dis_ref):
    dis_ref[...] = lax.rsqrt(1.0 + p_ref[0:1, :] + p_ref[1:2, :])
  return pl.pallas_call(
      body, out_shape=jax.ShapeDtypeStruct((1, n), jnp.float32))(deg_parts)


def _scale_matmul(x, w, dis_col, bn):
  n, k = x.shape
  d = w.shape[1]

  def body(x_ref, w_ref, dis_ref, y_ref):
    y_ref[...] = dis_ref[...] * jnp.dot(
        x_ref[...], w_ref[...], preferred_element_type=jnp.float32)

  return pl.pallas_call(
      body,
      grid=(n // bn,),
      in_specs=[
          pl.BlockSpec((bn, k), lambda i: (i, 0)),
          pl.BlockSpec((k, d), lambda i: (0, 0)),
          pl.BlockSpec((bn, 1), lambda i: (i, 0)),
      ],
      out_specs=pl.BlockSpec((bn, d), lambda i: (i, 0)),
      out_shape=jax.ShapeDtypeStruct((n, d), jnp.float32),
  )(x, w, dis_col)


def _mid_layer(s_parts, y1, dis_col, b1, w2, bn):
  n, d1 = y1.shape
  d2 = w2.shape[1]

  def body(s_ref, y1_ref, dis_ref, b1_ref, w2_ref, y2_ref):
    t = s_ref[0] + s_ref[1] + y1_ref[...]
    h = jnp.maximum(dis_ref[...] * t + b1_ref[...], 0.0)
    y2_ref[...] = dis_ref[...] * jnp.dot(
        h, w2_ref[...], preferred_element_type=jnp.float32)

  return pl.pallas_call(
      body,
      grid=(n // bn,),
      in_specs=[
          pl.BlockSpec((_NC, bn, d1), lambda i: (0, i, 0)),
          pl.BlockSpec((bn, d1), lambda i: (i, 0)),
          pl.BlockSpec((bn, 1), lambda i: (i, 0)),
          pl.BlockSpec((1, d1), lambda i: (0, 0)),
          pl.BlockSpec((d1, d2), lambda i: (0, 0)),
      ],
      out_specs=pl.BlockSpec((bn, d2), lambda i: (i, 0)),
      out_shape=jax.ShapeDtypeStruct((n, d2), jnp.float32),
  )(s_parts, y1, dis_col, b1, w2)


def _final_layer(s_parts, y2, dis_col, b2, bn):
  n, d2 = y2.shape

  def body(s_ref, y2_ref, dis_ref, b2_ref, z_ref):
    z_ref[...] = dis_ref[...] * (s_ref[0] + s_ref[1] + y2_ref[...]) \
        + b2_ref[...]

  return pl.pallas_call(
      body,
      grid=(n // bn,),
      in_specs=[
          pl.BlockSpec((_NC, bn, d2), lambda i: (0, i, 0)),
          pl.BlockSpec((bn, d2), lambda i: (i, 0)),
          pl.BlockSpec((bn, 1), lambda i: (i, 0)),
          pl.BlockSpec((1, d2), lambda i: (0, 0)),
      ],
      out_specs=pl.BlockSpec((bn, d2), lambda i: (i, 0)),
      out_shape=jax.ShapeDtypeStruct((n, d2), jnp.float32),
  )(s_parts, y2, dis_col, b2)


@jax.jit
def kernel(x, edge_index, W1, b1, W2, b2):
  n = x.shape[0]
  e = edge_index.shape[1]
  src = edge_index[0].astype(jnp.int32)
  dst = edge_index[1].astype(jnp.int32)

  align = _NS * _K * _NBUF
  n_super_tot = -(-e // align)
  e_pad = n_super_tot * align
  np_rows = n + _L
  if e_pad > e:
    src = jnp.concatenate([src, jnp.zeros((e_pad - e,), jnp.int32)])
    dst = jnp.concatenate([dst, jnp.full((e_pad - e,), n, jnp.int32)])
  src = src.reshape(e_pad // _K, _K)
  dst = dst.reshape(e_pad // _K, _K)

  sp_deg = (-(-n_super_tot * 31) // 40, None)
  sp_deg = (sp_deg[0], n_super_tot - sp_deg[0])
  sup_tot = n_super_tot * _NS
  sup0_d1 = (sup_tot * 72 + 50) // 100
  sup0_d2 = (sup_tot * 69 + 50) // 100
  sp_d1 = (sup0_d1, sup_tot - sup0_d1)
  sp_d2 = (sup0_d2, sup_tot - sup0_d2)

  bn = 2000
  deg_parts = _deg_kernel(n, np_rows, *sp_deg)(dst)
  dis_col = _dis(deg_parts, n).reshape(n, 1)

  y1 = _scale_matmul(x, W1, dis_col, bn)
  s1 = _msg_kernel(n, np_rows, y1.shape[1], *sp_d1)(y1, src, dst)
  y2 = _mid_layer(s1, y1, dis_col, b1.reshape(1, -1), W2, bn)
  s2 = _msg_kernel(n, np_rows, y2.shape[1], *sp_d2)(y2, src, dst)
  return _final_layer(s2, y2, dis_col, b2.reshape(1, -1), bn)


# speedup vs baseline: 1.0816x; 1.0123x over previous
import functools

import jax
import jax.numpy as jnp
from jax import lax
from jax.experimental import pallas as pl
from jax.experimental.pallas import tpu as pltpu
from jax.experimental.pallas import tpu_sc as plsc

_L = 16
_K = 128
_NC = 2
_NS = 16
_NW = _NC * _NS


_NBUF = 4


def _deg_kernel(n, np_rows, ns0, ns1):
  mesh = plsc.VectorSubcoreMesh(core_axis_name="c", subcore_axis_name="s")
  nc0, nc1 = ns0 * _NBUF, ns1 * _NBUF
  nc_max = max(nc0, nc1)

  @functools.partial(
      pl.kernel, mesh=mesh,
      out_type=jax.ShapeDtypeStruct((_NC, n), jnp.float32),
      compiler_params=pltpu.CompilerParams(use_tc_tiling_on_sc=False),
      scratch_types=[
          pltpu.VMEM((nc_max, _K), jnp.int32),
          pltpu.VMEM((_K,), jnp.float32),
          pltpu.VMEM((np_rows,), jnp.float32),
          pltpu.VMEM_SHARED((np_rows,), jnp.float32),
          pltpu.SemaphoreType.DMA,
      ] + [pltpu.SemaphoreType.DMA] * _NBUF)
  def degk(dst_hbm, out_hbm, idx_d, ones_v, zbuf, acc, sem_i, *sem_sc):
    c = lax.axis_index("c")
    s = lax.axis_index("s")

    @pl.when(c == 0)
    def _stage0():
      pltpu.async_copy(dst_hbm.at[pl.ds(s * nc0, nc0), :],
                       idx_d.at[pl.ds(0, nc0), :], sem_i)

    @pl.when(c == 1)
    def _stage1():
      pltpu.async_copy(dst_hbm.at[pl.ds(_NS * nc0 + s * nc1, nc1), :],
                       idx_d.at[pl.ds(0, nc1), :], sem_i)

    for j in range(_K // _L):
      ones_v[pl.ds(j * _L, _L)] = jnp.ones((_L,), jnp.float32)

    @pl.when(s == 0)
    def _zero():
      def zi(i, carry):
        zbuf[pl.ds(i * _L, _L)] = jnp.zeros((_L,), jnp.float32)
        return carry
      lax.fori_loop(0, np_rows // _L, zi, 0)
      pltpu.sync_copy(zbuf, acc)

    @pl.when(c == 0)
    def _wait0():
      pltpu.make_async_copy(dst_hbm.at[pl.ds(s * nc0, nc0), :],
                            idx_d.at[pl.ds(0, nc0), :], sem_i).wait()

    @pl.when(c == 1)
    def _wait1():
      pltpu.make_async_copy(dst_hbm.at[pl.ds(0, nc1), :],
                            idx_d.at[pl.ds(0, nc1), :], sem_i).wait()
    plsc.subcore_barrier()

    n_super = jnp.where(c == 0, ns0, ns1)

    def body(gs, carry):
      for b in range(_NBUF):
        g = gs * _NBUF + b

        @pl.when(gs > 0)
        def _drain():
          pltpu.make_async_copy(ones_v, acc.at[idx_d.at[g]],
                                sem_sc[b]).wait()
        pltpu.async_copy(ones_v, acc.at[idx_d.at[g]], sem_sc[b], add=True)
      return carry
    lax.fori_loop(0, n_super, body, 0)
    for b in range(_NBUF):
      g = (n_super - 1) * _NBUF + b
      pltpu.make_async_copy(ones_v, acc.at[idx_d.at[g]], sem_sc[b]).wait()

    plsc.subcore_barrier()

    @pl.when(s == 0)
    def _out():
      pltpu.sync_copy(acc.at[pl.ds(0, n)], out_hbm.at[c])

  return degk


def _msg_kernel(n, np_rows, d, sup0, sup1):
  mesh = plsc.VectorSubcoreMesh(core_axis_name="c", subcore_axis_name="s")
  zrows = np_rows // _NS
  orows = n // _NS
  zspan = _NBUF * _K

  @functools.partial(
      pl.kernel, mesh=mesh,
      out_type=jax.ShapeDtypeStruct((_NC, n, d), jnp.float32),
      compiler_params=pltpu.CompilerParams(use_tc_tiling_on_sc=False),
      scratch_types=[
          pltpu.VMEM((2, _NBUF, _K), jnp.int32),
          pltpu.VMEM((2, _NBUF, _K), jnp.int32),
          pltpu.VMEM((2, _NBUF, _K, d), jnp.float32),
          pltpu.VMEM_SHARED((np_rows, d), jnp.float32),
          pltpu.SMEM((1,), jnp.int32),
          pltpu.SemaphoreType.DMA,
          pltpu.SemaphoreType.DMA,
      ] + [pltpu.SemaphoreType.DMA] * (4 * _NBUF))
  def msgk(y_hbm, src_hbm, dst_hbm, out_hbm, idx_s, idx_d, rows_v, acc, cnt,
           *sems):
    sem_i = sems[:2]
    sem_g = (sems[2:2 + _NBUF], sems[2 + _NBUF:2 + 2 * _NBUF])
    sem_sc = (sems[2 + 2 * _NBUF:2 + 3 * _NBUF], sems[2 + 3 * _NBUF:])
    c = lax.axis_index("c")
    s = lax.axis_index("s")
    n_sup = jnp.where(c == 0, sup0, sup1)
    shard0 = jnp.where(c == 0, 0, sup0)

    @pl.when(s == 0)
    def _cinit():
      cnt[0] = 0

    with jax.named_scope("zfill"):
      def zi(i, carry):
        for j in range(d // _L):
          rows_v[0, i // _K, i % _K, pl.ds(j * _L, _L)] = (
              jnp.zeros((_L,), jnp.float32))
        return carry
      lax.fori_loop(0, min(zspan, zrows), zi, 0)
    with jax.named_scope("zdma"):
      done = 0
      while done < zrows:
        step = min(_K, zrows - done)
        pltpu.sync_copy(
            rows_v.at[0, done // _K % _NBUF, pl.ds(0, step), :],
            acc.at[pl.ds(s * zrows + done, step), :])
        done += step
    plsc.subcore_barrier()

    def fire_idx(p, sup):
      chunk0 = (shard0 + sup) * _NBUF
      pltpu.async_copy(src_hbm.at[pl.ds(chunk0, _NBUF), :], idx_s.at[p],
                       sem_i[p])
      pltpu.async_copy(dst_hbm.at[pl.ds(chunk0, _NBUF), :], idx_d.at[p],
                       sem_i[p])

    def wait_idx(p):
      pltpu.make_async_copy(src_hbm.at[pl.ds(0, _NBUF), :], idx_s.at[p],
                            sem_i[p]).wait()
      pltpu.make_async_copy(dst_hbm.at[pl.ds(0, _NBUF), :], idx_d.at[p],
                            sem_i[p]).wait()

    with jax.named_scope("edges"):
      s_init = []
      for p in range(2):
        sp = plsc.fetch_and_add(cnt.at[0], 1, subcore_id=0)

        @pl.when(sp < n_sup)
        def _prime(p=p, sp=sp):
          fire_idx(p, sp)
        s_init.append(sp)

      def body(_, carry):
        new = []
        for p in range(2):
          sp = carry[p]
          active = sp < n_sup

          @pl.when(active)
          def _visit(p=p):
            wait_idx(p)
            for b in range(_NBUF):
              pltpu.async_copy(y_hbm.at[idx_s.at[p, b]], rows_v.at[p, b],
                               sem_g[p][b])
            for b in range(_NBUF):
              pltpu.make_async_copy(y_hbm.at[idx_s.at[p, b]],
                                    rows_v.at[p, b], sem_g[p][b]).wait()
              pltpu.async_copy(rows_v.at[p, b], acc.at[idx_d.at[p, b]],
                               sem_sc[p][b], add=True)
            for b in range(_NBUF):
              pltpu.make_async_copy(rows_v.at[p, b], acc.at[idx_d.at[p, b]],
                                    sem_sc[p][b]).wait()
          nxt = lax.cond(
              active,
              lambda: plsc.fetch_and_add(cnt.at[0], 1, subcore_id=0),
              lambda sp=sp: sp)

          @pl.when(active & (nxt < n_sup))
          def _prefetch(p=p, nxt=nxt):
            fire_idx(p, nxt)
          new.append(nxt)
        return tuple(new)
      lax.fori_loop(0, n_sup // 2 + 2, body, tuple(s_init))

    with jax.named_scope("endbar"):
      plsc.subcore_barrier()
    with jax.named_scope("copyout"):
      pltpu.sync_copy(acc.at[pl.ds(s * orows, orows), :],
                      out_hbm.at[c, pl.ds(s * orows, orows), :])

  return msgk


def _dis(deg_parts, n):
  def body(p_ref, dis_ref):
    dis_ref[...] = lax.rsqrt(1.0 + p_ref[0:1, :] + p_ref[1:2, :])
  return pl.pallas_call(
      body, out_shape=jax.ShapeDtypeStruct((1, n), jnp.float32))(deg_parts)


def _scale_matmul(x, w, dis_col, bn):
  n, k = x.shape
  d = w.shape[1]

  def body(x_ref, w_ref, dis_ref, y_ref):
    y_ref[...] = dis_ref[...] * jnp.dot(
        x_ref[...], w_ref[...], preferred_element_type=jnp.float32)

  return pl.pallas_call(
      body,
      grid=(n // bn,),
      in_specs=[
          pl.BlockSpec((bn, k), lambda i: (i, 0)),
          pl.BlockSpec((k, d), lambda i: (0, 0)),
          pl.BlockSpec((bn, 1), lambda i: (i, 0)),
      ],
      out_specs=pl.BlockSpec((bn, d), lambda i: (i, 0)),
      out_shape=jax.ShapeDtypeStruct((n, d), jnp.float32),
  )(x, w, dis_col)


def _mid_layer(s_parts, y1, dis_col, b1, w2, bn):
  n, d1 = y1.shape
  d2 = w2.shape[1]

  def body(s_ref, y1_ref, dis_ref, b1_ref, w2_ref, y2_ref):
    t = s_ref[0] + s_ref[1] + y1_ref[...]
    h = jnp.maximum(dis_ref[...] * t + b1_ref[...], 0.0)
    y2_ref[...] = dis_ref[...] * jnp.dot(
        h, w2_ref[...], preferred_element_type=jnp.float32)

  return pl.pallas_call(
      body,
      grid=(n // bn,),
      in_specs=[
          pl.BlockSpec((_NC, bn, d1), lambda i: (0, i, 0)),
          pl.BlockSpec((bn, d1), lambda i: (i, 0)),
          pl.BlockSpec((bn, 1), lambda i: (i, 0)),
          pl.BlockSpec((1, d1), lambda i: (0, 0)),
          pl.BlockSpec((d1, d2), lambda i: (0, 0)),
      ],
      out_specs=pl.BlockSpec((bn, d2), lambda i: (i, 0)),
      out_shape=jax.ShapeDtypeStruct((n, d2), jnp.float32),
  )(s_parts, y1, dis_col, b1, w2)


def _final_layer(s_parts, y2, dis_col, b2, bn):
  n, d2 = y2.shape

  def body(s_ref, y2_ref, dis_ref, b2_ref, z_ref):
    z_ref[...] = dis_ref[...] * (s_ref[0] + s_ref[1] + y2_ref[...]) \
        + b2_ref[...]

  return pl.pallas_call(
      body,
      grid=(n // bn,),
      in_specs=[
          pl.BlockSpec((_NC, bn, d2), lambda i: (0, i, 0)),
          pl.BlockSpec((bn, d2), lambda i: (i, 0)),
          pl.BlockSpec((bn, 1), lambda i: (i, 0)),
          pl.BlockSpec((1, d2), lambda i: (0, 0)),
      ],
      out_specs=pl.BlockSpec((bn, d2), lambda i: (i, 0)),
      out_shape=jax.ShapeDtypeStruct((n, d2), jnp.float32),
  )(s_parts, y2, dis_col, b2)


@jax.jit
def kernel(x, edge_index, W1, b1, W2, b2):
  n = x.shape[0]
  e = edge_index.shape[1]
  src = edge_index[0].astype(jnp.int32)
  dst = edge_index[1].astype(jnp.int32)

  align = _NS * _K * _NBUF
  n_super_tot = -(-e // align)
  e_pad = n_super_tot * align
  np_rows = n + _L
  if e_pad > e:
    src = jnp.concatenate([src, jnp.zeros((e_pad - e,), jnp.int32)])
    dst = jnp.concatenate([dst, jnp.full((e_pad - e,), n, jnp.int32)])
  src = src.reshape(e_pad // _K, _K)
  dst = dst.reshape(e_pad // _K, _K)

  sp_deg = (-(-n_super_tot * 31) // 40, None)
  sp_deg = (sp_deg[0], n_super_tot - sp_deg[0])
  sup_tot = n_super_tot * _NS
  sup0_d1 = (sup_tot * 78 + 50) // 100
  sup0_d2 = (sup_tot * 74 + 50) // 100
  sp_d1 = (sup0_d1, sup_tot - sup0_d1)
  sp_d2 = (sup0_d2, sup_tot - sup0_d2)

  bn = 2000
  deg_parts = _deg_kernel(n, np_rows, *sp_deg)(dst)
  dis_col = _dis(deg_parts, n).reshape(n, 1)

  y1 = _scale_matmul(x, W1, dis_col, bn)
  s1 = _msg_kernel(n, np_rows, y1.shape[1], *sp_d1)(y1, src, dst)
  y2 = _mid_layer(s1, y1, dis_col, b1.reshape(1, -1), W2, bn)
  s2 = _msg_kernel(n, np_rows, y2.shape[1], *sp_d2)(y2, src, dst)
  return _final_layer(s2, y2, dis_col, b2.reshape(1, -1), bn)
